# baseline (speedup 1.0000x reference)
import functools

import jax
import jax.numpy as jnp
from jax.experimental import pallas as pl
from jax.experimental.pallas import tpu as pltpu

DELTA_V = 0.5
DELTA_D = 1.5
ALPHA = 1.0
BETA = 1.0
GAMMA = 0.001
MAX_VIEWS = 100
NUM_CLASSES = 19
HB = 8
HBLK = 64


def _body(fa_ref, fb_ref, l_ref, out_ref, sums_ref, cnt_ref, means_ref,
          coef_ref, mexp_ref, coefexp_ref, accv_ref, *, nb, nbatch, d, w):
    p = pl.program_id(0)
    i = pl.program_id(1)
    C = NUM_CLASSES
    CH = C * HB
    DH = d * HB

    def onehot2(lb):
        labrep = jnp.broadcast_to(lb[None], (C, HB, w)).reshape(CH, w)
        cls = jax.lax.broadcasted_iota(jnp.int32, (CH, w), 0) // HB
        return (labrep == cls).astype(jnp.float32)

    @pl.when((p == 0) & (i == 0))
    def _init():
        sums_ref[...] = jnp.zeros_like(sums_ref)
        cnt_ref[...] = jnp.zeros_like(cnt_ref)

    @pl.when(p == 0)
    def _accum_sums():
        ones = jnp.ones((1, w), jnp.float32)
        sacc = sums_ref[...]
        cacc = cnt_ref[...]
        for b in range(nbatch):
            for s in range(HBLK // HB):
                oh2 = onehot2(l_ref[b, s * HB:(s + 1) * HB])
                fa2 = fa_ref[b][:, s * HB:(s + 1) * HB, :].reshape(DH // 2, w)
                fb2 = fb_ref[b][:, s * HB:(s + 1) * HB, :].reshape(DH // 2, w)
                sacc = sacc + jnp.concatenate([
                    jax.lax.dot_general(
                        oh2, fa2, (((1,), (1,)), ((), ())),
                        preferred_element_type=jnp.float32),
                    jax.lax.dot_general(
                        oh2, fb2, (((1,), (1,)), ((), ())),
                        preferred_element_type=jnp.float32)], axis=1)
                cacc = cacc + jax.lax.dot_general(
                    oh2, ones, (((1,), (1,)), ((), ())),
                    preferred_element_type=jnp.float32)
        sums_ref[...] = sacc
        cnt_ref[...] = cacc

    @pl.when((p == 1) & (i == 0))
    def _prep():
        rx = jax.lax.broadcasted_iota(jnp.int32, (CH, DH), 0) % HB
        ry = jax.lax.broadcasted_iota(jnp.int32, (CH, DH), 1) % HB
        masked = sums_ref[...] * (rx == ry).astype(jnp.float32)
        a_cx = (jax.lax.broadcasted_iota(jnp.int32, (C, CH), 1) // HB
                == jax.lax.broadcasted_iota(jnp.int32, (C, CH), 0)
                ).astype(jnp.float32)
        b_yd = (jax.lax.broadcasted_iota(jnp.int32, (DH, d), 0) // HB
                == jax.lax.broadcasted_iota(jnp.int32, (DH, d), 1)
                ).astype(jnp.float32)
        sums = jax.lax.dot_general(
            jax.lax.dot_general(a_cx, masked, (((1,), (0,)), ((), ())),
                                preferred_element_type=jnp.float32),
            b_yd, (((1,), (0,)), ((), ())),
            preferred_element_type=jnp.float32)
        cnt = jax.lax.dot_general(
            a_cx, cnt_ref[...], (((1,), (0,)), ((), ())),
            preferred_element_type=jnp.float32)
        safe = jnp.maximum(cnt, 1.0)
        m = sums / safe
        means_ref[...] = m
        msq = jnp.sum(m * m, axis=1, keepdims=True)
        valid = (cnt > float(MAX_VIEWS)).astype(jnp.float32)
        coef_ref[...] = jnp.concatenate(
            [valid / safe, msq, valid, cnt], axis=1)
        x1 = (jax.lax.broadcasted_iota(jnp.int32, (CH, C), 0) // HB
              == jax.lax.broadcasted_iota(jnp.int32, (CH, C), 1)
              ).astype(jnp.float32)
        mexp_ref[...] = jax.lax.dot_general(
            jax.lax.dot_general(x1, -2.0 * m, (((1,), (0,)), ((), ())),
                                preferred_element_type=jnp.float32),
            jnp.transpose(b_yd), (((1,), (0,)), ((), ())),
            preferred_element_type=jnp.float32) * (rx == ry).astype(
                jnp.float32)
        coefexp_ref[...] = jax.lax.dot_general(
            x1, jnp.concatenate([msq, valid / safe], axis=1),
            (((1,), (0,)), ((), ())),
            preferred_element_type=jnp.float32)
        accv_ref[...] = jnp.zeros_like(accv_ref)

    @pl.when(p == 1)
    def _accum_var():
        mexp = mexp_ref[...]
        ce = coefexp_ref[...]
        msqe = ce[:, 0:1]
        cvare = ce[:, 1:2]
        acc = accv_ref[...]
        for b in range(nbatch):
            for s in range(HBLK // HB):
                oh2 = onehot2(l_ref[b, s * HB:(s + 1) * HB])
                fha = fa_ref[b][:, s * HB:(s + 1) * HB, :]
                fhb = fb_ref[b][:, s * HB:(s + 1) * HB, :]
                fa2 = fha.reshape(DH // 2, w)
                fb2 = fhb.reshape(DH // 2, w)
                q2 = (jax.lax.dot_general(
                          mexp[:, :DH // 2], fa2, (((1,), (0,)), ((), ())),
                          preferred_element_type=jnp.float32)
                      + jax.lax.dot_general(
                          mexp[:, DH // 2:], fb2, (((1,), (0,)), ((), ())),
                          preferred_element_type=jnp.float32))
                selq = jnp.sum((oh2 * (q2 + msqe)).reshape(C, HB, w),
                               axis=0)
                selc = jnp.sum((oh2 * cvare).reshape(C, HB, w), axis=0)
                fsq = (jnp.sum(fha * fha, axis=0)
                       + jnp.sum(fhb * fhb, axis=0))
                dist = jnp.sqrt(jnp.maximum(fsq + selq, 0.0))
                h = jnp.maximum(dist - DELTA_V, 0.0)
                acc = acc + h * h * selc
        accv_ref[...] = acc

    @pl.when((p == 1) & (i == nb - 1))
    def _final():
        m = means_ref[...]
        coef = coef_ref[...]
        msq = coef[:, 1:2]
        valid = coef[:, 2:3]
        total = jnp.sum(valid, keepdims=True)
        G = jax.lax.dot_general(
            m, m, (((1,), (1,)), ((), ())),
            preferred_element_type=jnp.float32)
        d2 = jnp.maximum(msq + jnp.transpose(msq) - 2.0 * G, 0.0)
        dd = jnp.maximum(2.0 * DELTA_D - jnp.sqrt(d2), 0.0)
        iota = jax.lax.broadcasted_iota(jnp.int32, (C, 1), 0).astype(
            jnp.float32)
        lastv = jnp.max(jnp.where(valid > 0.0, iota, -1.0), keepdims=True)
        bmask = valid * (iota != lastv).astype(jnp.float32)
        wmat = valid * jnp.transpose(bmask)
        loss_dist = jnp.sum(wmat * dd * dd, keepdims=True)
        loss_reg = jnp.sum(valid * jnp.sqrt(msq), keepdims=True)
        loss_var = jnp.sum(accv_ref[...], keepdims=True)
        out = (ALPHA * loss_var / total
               + BETA * loss_dist / (total * (total - 1.0))
               + GAMMA * loss_reg / total)
        out_ref[...] = out


def kernel(feats, labels):
    B, D, H, W = feats.shape
    nb = H // HBLK
    C = NUM_CLASSES
    out = pl.pallas_call(
        functools.partial(_body, nb=nb, nbatch=B, d=D, w=W),
        grid=(2, nb),
        in_specs=[
            pl.BlockSpec((B, D // 2, HBLK, W), lambda p, i: (0, 0, i, 0)),
            pl.BlockSpec((B, D // 2, HBLK, W), lambda p, i: (0, 1, i, 0)),
            pl.BlockSpec((B, HBLK, W), lambda p, i: (0, i, 0)),
        ],
        out_specs=pl.BlockSpec((1, 1), lambda p, i: (0, 0)),
        out_shape=jax.ShapeDtypeStruct((1, 1), jnp.float32),
        scratch_shapes=[
            pltpu.VMEM((C * HB, D * HB), jnp.float32),
            pltpu.VMEM((C * HB, 1), jnp.float32),
            pltpu.VMEM((C, D), jnp.float32),
            pltpu.VMEM((C, 4), jnp.float32),
            pltpu.VMEM((C * HB, D * HB), jnp.float32),
            pltpu.VMEM((C * HB, 2), jnp.float32),
            pltpu.VMEM((HB, W), jnp.float32),
        ],
        compiler_params=pltpu.CompilerParams(
            dimension_semantics=("arbitrary", "arbitrary")),
    )(feats, feats, labels)
    return out[0, 0]


# speedup vs baseline: 1.0385x; 1.0385x over previous
import functools

import jax
import jax.numpy as jnp
from jax.experimental import pallas as pl
from jax.experimental.pallas import tpu as pltpu

DELTA_V = 0.5
DELTA_D = 1.5
ALPHA = 1.0
BETA = 1.0
GAMMA = 0.001
MAX_VIEWS = 100
NUM_CLASSES = 19
HB = 8
HBLK = 64


def _body(f_ref, l_ref, out_ref, sums_ref, cnt_ref, means_ref, coef_ref,
          mexp_ref, coefexp_ref, accv_ref, *, nb, nbatch, d, w):
    p = pl.program_id(0)
    i = pl.program_id(1)
    C = NUM_CLASSES
    CH = C * HB
    DH = d * HB

    def onehot2(lb):
        labrep = jnp.broadcast_to(lb[None], (C, HB, w)).reshape(CH, w)
        cls = jax.lax.broadcasted_iota(jnp.int32, (CH, w), 0) // HB
        return (labrep == cls).astype(jnp.float32)

    @pl.when((p == 0) & (i == 0))
    def _init():
        sums_ref[...] = jnp.zeros_like(sums_ref)
        cnt_ref[...] = jnp.zeros_like(cnt_ref)

    @pl.when(p == 0)
    def _accum_sums():
        ones = jnp.ones((1, w), jnp.float32)
        sacc = sums_ref[...]
        cacc = cnt_ref[...]
        for b in range(nbatch):
            for s in range(HBLK // HB):
                oh2 = onehot2(l_ref[b, s * HB:(s + 1) * HB])
                fb2 = f_ref[b][:, s * HB:(s + 1) * HB, :].reshape(DH, w)
                sacc = sacc + jax.lax.dot_general(
                    oh2, fb2, (((1,), (1,)), ((), ())),
                    preferred_element_type=jnp.float32)
                cacc = cacc + jax.lax.dot_general(
                    oh2, ones, (((1,), (1,)), ((), ())),
                    preferred_element_type=jnp.float32)
        sums_ref[...] = sacc
        cnt_ref[...] = cacc

    @pl.when((p == 1) & (i == 0))
    def _prep():
        rx = jax.lax.broadcasted_iota(jnp.int32, (CH, DH), 0) % HB
        ry = jax.lax.broadcasted_iota(jnp.int32, (CH, DH), 1) % HB
        masked = sums_ref[...] * (rx == ry).astype(jnp.float32)
        a_cx = (jax.lax.broadcasted_iota(jnp.int32, (C, CH), 1) // HB
                == jax.lax.broadcasted_iota(jnp.int32, (C, CH), 0)
                ).astype(jnp.float32)
        b_yd = (jax.lax.broadcasted_iota(jnp.int32, (DH, d), 0) // HB
                == jax.lax.broadcasted_iota(jnp.int32, (DH, d), 1)
                ).astype(jnp.float32)
        sums = jax.lax.dot_general(
            jax.lax.dot_general(a_cx, masked, (((1,), (0,)), ((), ())),
                                preferred_element_type=jnp.float32),
            b_yd, (((1,), (0,)), ((), ())),
            preferred_element_type=jnp.float32)
        cnt = jax.lax.dot_general(
            a_cx, cnt_ref[...], (((1,), (0,)), ((), ())),
            preferred_element_type=jnp.float32)
        safe = jnp.maximum(cnt, 1.0)
        m = sums / safe
        means_ref[...] = m
        msq = jnp.sum(m * m, axis=1, keepdims=True)
        valid = (cnt > float(MAX_VIEWS)).astype(jnp.float32)
        coef_ref[...] = jnp.concatenate(
            [valid / safe, msq, valid, cnt], axis=1)
        x1 = (jax.lax.broadcasted_iota(jnp.int32, (CH, C), 0) // HB
              == jax.lax.broadcasted_iota(jnp.int32, (CH, C), 1)
              ).astype(jnp.float32)
        mexp_ref[...] = jax.lax.dot_general(
            jax.lax.dot_general(x1, -2.0 * m, (((1,), (0,)), ((), ())),
                                preferred_element_type=jnp.float32),
            jnp.transpose(b_yd), (((1,), (0,)), ((), ())),
            preferred_element_type=jnp.float32) * (rx == ry).astype(
                jnp.float32)
        coefexp_ref[...] = jax.lax.dot_general(
            x1, jnp.concatenate([msq, valid / safe], axis=1),
            (((1,), (0,)), ((), ())),
            preferred_element_type=jnp.float32)
        accv_ref[...] = jnp.zeros_like(accv_ref)

    @pl.when(p == 1)
    def _accum_var():
        mexp = mexp_ref[...]
        ce = coefexp_ref[...]
        msqe = ce[:, 0:1]
        cvare = ce[:, 1:2]
        acc = accv_ref[...]
        for b in range(nbatch):
            for s in range(HBLK // HB):
                oh2 = onehot2(l_ref[b, s * HB:(s + 1) * HB])
                fb = f_ref[b][:, s * HB:(s + 1) * HB, :]
                fb2 = fb.reshape(DH, w)
                q2 = jax.lax.dot_general(
                    mexp, fb2, (((1,), (0,)), ((), ())),
                    preferred_element_type=jnp.float32)
                selq = jnp.sum((oh2 * (q2 + msqe)).reshape(C, HB, w),
                               axis=0)
                selc = jnp.sum((oh2 * cvare).reshape(C, HB, w), axis=0)
                fsq = jnp.sum(fb * fb, axis=0)
                dist = jnp.sqrt(jnp.maximum(fsq + selq, 0.0))
                h = jnp.maximum(dist - DELTA_V, 0.0)
                acc = acc + h * h * selc
        accv_ref[...] = acc

    @pl.when((p == 1) & (i == nb - 1))
    def _final():
        m = means_ref[...]
        coef = coef_ref[...]
        msq = coef[:, 1:2]
        valid = coef[:, 2:3]
        total = jnp.sum(valid, keepdims=True)
        G = jax.lax.dot_general(
            m, m, (((1,), (1,)), ((), ())),
            preferred_element_type=jnp.float32)
        d2 = jnp.maximum(msq + jnp.transpose(msq) - 2.0 * G, 0.0)
        dd = jnp.maximum(2.0 * DELTA_D - jnp.sqrt(d2), 0.0)
        iota = jax.lax.broadcasted_iota(jnp.int32, (C, 1), 0).astype(
            jnp.float32)
        lastv = jnp.max(jnp.where(valid > 0.0, iota, -1.0), keepdims=True)
        bmask = valid * (iota != lastv).astype(jnp.float32)
        wmat = valid * jnp.transpose(bmask)
        loss_dist = jnp.sum(wmat * dd * dd, keepdims=True)
        loss_reg = jnp.sum(valid * jnp.sqrt(msq), keepdims=True)
        loss_var = jnp.sum(accv_ref[...], keepdims=True)
        out = (ALPHA * loss_var / total
               + BETA * loss_dist / (total * (total - 1.0))
               + GAMMA * loss_reg / total)
        out_ref[...] = out


def kernel(feats, labels):
    B, D, H, W = feats.shape
    nb = H // HBLK
    C = NUM_CLASSES
    out = pl.pallas_call(
        functools.partial(_body, nb=nb, nbatch=B, d=D, w=W),
        grid=(2, nb),
        in_specs=[
            pl.BlockSpec((B, D, HBLK, W), lambda p, i: (0, 0, i, 0)),
            pl.BlockSpec((B, HBLK, W), lambda p, i: (0, i, 0)),
        ],
        out_specs=pl.BlockSpec((1, 1), lambda p, i: (0, 0)),
        out_shape=jax.ShapeDtypeStruct((1, 1), jnp.float32),
        scratch_shapes=[
            pltpu.VMEM((C * HB, D * HB), jnp.float32),
            pltpu.VMEM((C * HB, 1), jnp.float32),
            pltpu.VMEM((C, D), jnp.float32),
            pltpu.VMEM((C, 4), jnp.float32),
            pltpu.VMEM((C * HB, D * HB), jnp.float32),
            pltpu.VMEM((C * HB, 2), jnp.float32),
            pltpu.VMEM((HB, W), jnp.float32),
        ],
        compiler_params=pltpu.CompilerParams(
            dimension_semantics=("arbitrary", "arbitrary")),
    )(feats, labels)
    return out[0, 0]
